# baseline (speedup 1.0000x reference)
import functools

import jax
import jax.numpy as jnp
from jax import lax
from jax.experimental import pallas as pl
from jax.experimental.pallas import tpu as pltpu
from jax.experimental.pallas import tpu_sc as plsc

VOCAB = 1000000
D = 64
L = 50
LC = 51
NG = 104
NLANE = 16
NC = 2
NS = 16
NW = NC * NS
BATCH = 16384
PAIRS = BATCH // 2
PPT = PAIRS // NW


def _shuf(x, perm):
    return x.at[perm].get(mode="promise_in_bounds")


def _splat_sum(x, lane):
    for r in (8, 4, 2, 1):
        x = x + _shuf(x, lane ^ r)
    return x


def _combine(R, rbase, hoff, out_v, prow, ocol):
    def row(k, q):
        start = pl.multiple_of(hoff(rbase + k) + NLANE * q, NLANE)
        return R[rbase + k, pl.ds(start, NLANE)]

    c = [row(L, q) for q in range(4)]
    lane = lax.iota(jnp.int32, NLANE)
    zero = jnp.zeros((NLANE,), jnp.float32)
    acc = [zero] * 4
    wacc = zero
    for k in range(L):
        x = [row(k, q) for q in range(4)]
        s = None
        for q in range(4):
            d = x[q] - c[q]
            s = d * d if s is None else s + d * d
        w = jnp.exp(_splat_sum(s, lane) * -0.5)
        wacc = wacc + w
        for q in range(4):
            acc[q] = acc[q] + w * x[q]
    inv = 1.0 / (wacc + 1e-8)
    for q in range(4):
        out_v[prow, pl.ds(ocol + NLANE * q, NLANE)] = acc[q] * inv


@functools.partial(
    pl.kernel,
    out_type=jax.ShapeDtypeStruct((PAIRS, 2 * D), jnp.float32),
    mesh=plsc.VectorSubcoreMesh(core_axis_name="c", subcore_axis_name="s"),
    scratch_types=[
        pltpu.VMEM((PPT, 128), jnp.int32),
        pltpu.VMEM((PPT, 2 * D), jnp.float32),
        pltpu.VMEM((NG, 2 * D), jnp.float32),
        pltpu.VMEM((NG, 2 * D), jnp.float32),
        pltpu.SemaphoreType.DMA,
        pltpu.SemaphoreType.DMA,
    ],
)
def _krembed(idx_hbm, table_hbm, out_hbm, idx_v, out_v, bufA, bufB,
             semA, semB):
    wid = lax.axis_index("s") * NC + lax.axis_index("c")
    base = wid * PPT
    pltpu.sync_copy(idx_hbm.at[pl.ds(base, PPT)], idx_v)

    def fire(p, buf, sem):
        pltpu.async_copy(table_hbm.at[idx_v.at[p, pl.ds(0, NG)]], buf, sem)

    def drain(buf, sem):
        pltpu.make_async_copy(table_hbm.at[pl.ds(0, NG)], buf, sem).wait()

    def combine2(buf, p):
        hv = idx_v[p, pl.ds(96, NLANE)]
        hw = [hv[8], hv[9], hv[10], hv[11]]

        def hoff(j02):
            return ((hw[j02 // 32] >> (j02 % 32)) & 1) << 6

        _combine(buf, 0, hoff, out_v, p, 0)
        _combine(buf, LC, hoff, out_v, p, D)

    fire(0, bufA, semA)

    def body(i, carry):
        p = i * 2
        fire(p + 1, bufB, semB)
        drain(bufA, semA)
        combine2(bufA, p)

        @pl.when(p + 2 < PPT)
        def _():
            fire(p + 2, bufA, semA)

        drain(bufB, semB)
        combine2(bufB, p + 1)
        return carry

    lax.fori_loop(0, PPT // 2, body, 0)
    pltpu.sync_copy(out_v, out_hbm.at[pl.ds(base, PPT)])


def kernel(context, center, embedding_weights):
    i102 = jnp.concatenate([context, center[:, None]],
                           axis=1).reshape(PAIRS, 2 * LC)
    slots = jnp.pad(i102 >> 1, ((0, 0), (0, NG - 2 * LC)))
    hbits = jnp.pad(i102 & 1, ((0, 0), (0, 128 - 2 * LC)))
    shift = jnp.arange(32, dtype=jnp.int32)[None, None, :]
    hw = jnp.sum(hbits.reshape(PAIRS, 4, 32) << shift,
                 axis=2, dtype=jnp.int32)
    idx = jnp.concatenate(
        [slots, hw, jnp.zeros((PAIRS, 128 - NG - 4), jnp.int32)], axis=1)
    table2 = embedding_weights.reshape(VOCAB // 2, 2 * D)
    packed = _krembed(idx, table2)
    return packed.reshape(BATCH, D)
